# baseline (speedup 1.0000x reference)
import functools

import jax
import jax.numpy as jnp
from jax.experimental import pallas as pl
from jax.experimental.pallas import tpu as pltpu

F = 128


def _fused_body(nseg, ks, scale, *refs):
    a_refs = refs[0:nseg]
    x_refs = refs[nseg:2 * nseg]
    w_refs = refs[2 * nseg:3 * nseg]
    b_refs = refs[3 * nseg:4 * nseg]
    out_ref = refs[4 * nseg]
    h_ref = refs[4 * nseg + 1]

    m = pl.program_id(0)

    @pl.when(m == 0)
    def _():
        koff = 0
        for s in range(nseg):
            xb = x_refs[s][...]
            xc = jnp.concatenate([xb, xb * xb], axis=1)
            ht = jax.lax.dot_general(
                w_refs[s][...], xc, (((1,), (1,)), ((), ())),
                preferred_element_type=jnp.float32)
            h_ref[:, pl.ds(koff, ks[s])] = ht + b_refs[s][...]
            koff += ks[s]

    acc = None
    koff = 0
    for s in range(nseg):
        part = jax.lax.dot_general(
            h_ref[:, pl.ds(koff, ks[s])], a_refs[s][...],
            (((1,), (1,)), ((), ())),
            preferred_element_type=jnp.float32)
        acc = part if acc is None else acc + part
        koff += ks[s]

    y = scale * jnp.maximum(acc, 0.0)
    out_ref[...] = y.T


def _fused_output(a_list, x_list, w_list, b_list, scale, bm=256):
    nseg = len(a_list)
    m_rows = a_list[0].shape[0]
    ks = tuple(a.shape[1] for a in a_list)
    num_m = m_rows // bm

    b2_list = [b.reshape(F, 1) for b in b_list]

    a_specs = [pl.BlockSpec((bm, k), lambda mi: (mi, 0)) for k in ks]
    whole = lambda shape: pl.BlockSpec(shape, lambda mi: (0,) * len(shape))
    x_specs = [whole(x.shape) for x in x_list]
    w_specs = [whole(w.shape) for w in w_list]
    b_specs = [whole(b2.shape) for b2 in b2_list]
    out_spec = pl.BlockSpec((bm, F), lambda mi: (mi, 0))

    body = functools.partial(_fused_body, nseg, ks, scale)
    return pl.pallas_call(
        body,
        grid=(num_m,),
        in_specs=a_specs + x_specs + w_specs + b_specs,
        out_specs=out_spec,
        out_shape=jax.ShapeDtypeStruct((m_rows, F), jnp.float32),
        scratch_shapes=[pltpu.VMEM((F, sum(ks)), jnp.float32)],
        compiler_params=pltpu.CompilerParams(
            dimension_semantics=("arbitrary",)),
    )(*a_list, *x_list, *w_list, *b2_list)


def _seq_body(nseg, ks, num_m, bm, scale, *refs):
    a_refs = refs[0:nseg]
    x_refs = refs[nseg:2 * nseg]
    w_refs = refs[2 * nseg:3 * nseg]
    b_refs = refs[3 * nseg:4 * nseg]
    out_ref = refs[4 * nseg]
    acc_ref = refs[4 * nseg + 1]
    h_ref = refs[4 * nseg + 2]

    i = pl.program_id(0)
    koff = 0
    for s in range(nseg):
        first = s * num_m
        in_phase = (i >= first) & (i < first + num_m)

        @pl.when(in_phase & (i == first))
        def _(s=s, koff=koff):
            xb = x_refs[s][...]
            xc = jnp.concatenate([xb, xb * xb], axis=1)
            ht = jax.lax.dot_general(
                w_refs[s][...], xc, (((1,), (1,)), ((), ())),
                preferred_element_type=jnp.float32)
            h_ref[:, pl.ds(koff, ks[s])] = ht + b_refs[s][...]

        @pl.when(in_phase)
        def _(s=s, koff=koff, first=first):
            p = i - first
            part = jax.lax.dot_general(
                h_ref[:, pl.ds(koff, ks[s])], a_refs[s][...],
                (((1,), (1,)), ((), ())),
                preferred_element_type=jnp.float32)
            if s == 0:
                acc_ref[:, pl.ds(p * bm, bm)] = part
            elif s < nseg - 1:
                acc_ref[:, pl.ds(p * bm, bm)] += part
            else:
                tot = acc_ref[:, pl.ds(p * bm, bm)] + part
                y = scale * jnp.maximum(tot, 0.0)
                out_ref[...] = y.T

        koff += ks[s]


def _fused_output_seq(a_list, x_list, w_list, b_list, scale, bm=256):
    nseg = len(a_list)
    m_rows = a_list[0].shape[0]
    ks = tuple(a.shape[1] for a in a_list)
    num_m = m_rows // bm

    b2_list = [b.reshape(F, 1) for b in b_list]

    a_specs = []
    for s, k in enumerate(ks):
        def a_map(i, s=s):
            return (jnp.clip(i - s * num_m, 0, num_m - 1), 0)
        a_specs.append(pl.BlockSpec((bm, k), a_map))
    whole = lambda shape: pl.BlockSpec(shape, lambda i: (0,) * len(shape))
    x_specs = [whole(x.shape) for x in x_list]
    w_specs = [whole(w.shape) for w in w_list]
    b_specs = [whole(b2.shape) for b2 in b2_list]
    out_spec = pl.BlockSpec(
        (bm, F), lambda i: (jnp.clip(i - (nseg - 1) * num_m, 0, num_m - 1), 0))

    body = functools.partial(_seq_body, nseg, ks, num_m, bm, scale)
    return pl.pallas_call(
        body,
        grid=(nseg * num_m,),
        in_specs=a_specs + x_specs + w_specs + b_specs,
        out_specs=out_spec,
        out_shape=jax.ShapeDtypeStruct((m_rows, F), jnp.float32),
        scratch_shapes=[pltpu.VMEM((F, m_rows), jnp.float32),
                        pltpu.VMEM((F, sum(ks)), jnp.float32)],
        compiler_params=pltpu.CompilerParams(
            dimension_semantics=("arbitrary",)),
    )(*a_list, *x_list, *w_list, *b2_list)


def kernel(L0, L1, L2, D1invB1, D2B1TD1inv, B2TD2inv, B2D3, X0, X1, X2,
           Wn2n, bn2n, Wn2e, bn2e, We2e, be2e, We2n, be2n, We2t, be2t,
           Wt2e, bt2e, Wt2t, bt2t):
    Y0 = _fused_output_seq([L0, D1invB1], [X0, X1], [Wn2n, We2n], [bn2n, be2n],
                       0.5)
    Y1 = _fused_output_seq([L1, D2B1TD1inv, B2D3], [X1, X0, X2],
                       [We2e, Wn2e, Wt2e], [be2e, bn2e, bt2e], 1.0 / 3.0)
    Y2 = _fused_output_seq([L2, B2TD2inv], [X2, X1], [Wt2t, We2t], [bt2t, be2t],
                       0.5)
    return (Y0, Y1, Y2)


# speedup vs baseline: 1.1810x; 1.1810x over previous
import functools

import jax
import jax.numpy as jnp
from jax.experimental import pallas as pl
from jax.experimental.pallas import tpu as pltpu

F = 128


def _fused_body(nseg, ks, scale, *refs):
    a_refs = refs[0:nseg]
    x_refs = refs[nseg:2 * nseg]
    w_refs = refs[2 * nseg:3 * nseg]
    b_refs = refs[3 * nseg:4 * nseg]
    out_ref = refs[4 * nseg]
    h_ref = refs[4 * nseg + 1]

    m = pl.program_id(0)

    @pl.when(m == 0)
    def _():
        koff = 0
        for s in range(nseg):
            xb = x_refs[s][...]
            xc = jnp.concatenate([xb, xb * xb], axis=1)
            ht = jax.lax.dot_general(
                w_refs[s][...], xc, (((1,), (1,)), ((), ())),
                preferred_element_type=jnp.float32)
            h_ref[:, pl.ds(koff, ks[s])] = ht + b_refs[s][...]
            koff += ks[s]

    acc = None
    koff = 0
    for s in range(nseg):
        part = jax.lax.dot_general(
            h_ref[:, pl.ds(koff, ks[s])], a_refs[s][...],
            (((1,), (1,)), ((), ())),
            preferred_element_type=jnp.float32)
        acc = part if acc is None else acc + part
        koff += ks[s]

    y = scale * jnp.maximum(acc, 0.0)
    out_ref[...] = y.T


def _fused_output(a_list, x_list, w_list, b_list, scale, bm=256):
    nseg = len(a_list)
    m_rows = a_list[0].shape[0]
    ks = tuple(a.shape[1] for a in a_list)
    num_m = m_rows // bm

    b2_list = [b.reshape(F, 1) for b in b_list]

    a_specs = [pl.BlockSpec((bm, k), lambda mi: (mi, 0)) for k in ks]
    whole = lambda shape: pl.BlockSpec(shape, lambda mi: (0,) * len(shape))
    x_specs = [whole(x.shape) for x in x_list]
    w_specs = [whole(w.shape) for w in w_list]
    b_specs = [whole(b2.shape) for b2 in b2_list]
    out_spec = pl.BlockSpec((bm, F), lambda mi: (mi, 0))

    body = functools.partial(_fused_body, nseg, ks, scale)
    return pl.pallas_call(
        body,
        grid=(num_m,),
        in_specs=a_specs + x_specs + w_specs + b_specs,
        out_specs=out_spec,
        out_shape=jax.ShapeDtypeStruct((m_rows, F), jnp.float32),
        scratch_shapes=[pltpu.VMEM((F, sum(ks)), jnp.float32)],
        compiler_params=pltpu.CompilerParams(
            dimension_semantics=("parallel",)),
    )(*a_list, *x_list, *w_list, *b2_list)


def kernel(L0, L1, L2, D1invB1, D2B1TD1inv, B2TD2inv, B2D3, X0, X1, X2,
           Wn2n, bn2n, Wn2e, bn2e, We2e, be2e, We2n, be2n, We2t, be2t,
           Wt2e, bt2e, Wt2t, bt2t):
    Y0 = _fused_output([L0, D1invB1], [X0, X1], [Wn2n, We2n], [bn2n, be2n],
                       0.5)
    Y1 = _fused_output([L1, D2B1TD1inv, B2D3], [X1, X0, X2],
                       [We2e, Wn2e, Wt2e], [be2e, bn2e, bt2e], 1.0 / 3.0)
    Y2 = _fused_output([L2, B2TD2inv], [X2, X1], [Wt2t, We2t], [bt2t, be2t],
                       0.5)
    return (Y0, Y1, Y2)
